# baseline (speedup 1.0000x reference)
import functools

import jax
import jax.numpy as jnp
from jax import lax
from jax.experimental import pallas as pl
from jax.experimental.pallas import tpu as pltpu
from jax.experimental.pallas import tpu_sc as plsc

B = 16384
L = 20
D = 32
N_EMB = 1000000
NC = 2
NS = 16
NW = NC * NS
ROWS_PER_W = B // NW
CB = 128
NCHUNK = ROWS_PER_W // CB
IDXW = 128
NGATHER = CB * L // IDXW

NJ_FULL = N_EMB // 128
NJ_MAIN = (NJ_FULL // NW) * NW
LAST_COLS = N_EMB - NJ_FULL * 128


def _relayout_body(src_hbm, tail_hbm, dst_hbm, slab_v, outq_v, sem):
    wid = lax.axis_index("s") * NC + lax.axis_index("c")
    lanes = lax.iota(jnp.int32, 16)

    def do_slab(j, _):
        pltpu.async_copy(src_hbm.at[:, pl.ds(j * 128, 128)],
                         slab_v, sem).wait()
        for g in range(8):
            rp = g * 16 + lanes
            rowv = lax.shift_right_logical(rp, 2)
            colbase = lax.shift_left(lax.bitwise_and(rp, 3), 5)
            for d in range(D):
                v = slab_v[d, pl.ds(g * 16, 16)]
                plsc.store_scatter(outq_v, [rowv, colbase + d], v)
        pltpu.async_copy(outq_v, dst_hbm.at[pl.ds(j * 32, 32), :],
                         sem).wait()
        return _

    lax.fori_loop(0, NJ_MAIN // NW, lambda t, c: do_slab(t * NW + wid, c),
                  0, unroll=False)
    @pl.when(wid < NJ_FULL - NJ_MAIN)
    def _():
        do_slab(NJ_MAIN + wid, 0)

    @pl.when(wid == NJ_FULL - NJ_MAIN)
    def _():
        pltpu.async_copy(tail_hbm, slab_v.at[pl.ds(0, 16), :], sem).wait()
        for g in range(LAST_COLS // 16):
            rp = g * 16 + lanes
            rowv = lax.shift_right_logical(rp, 2)
            colbase = lax.shift_left(lax.bitwise_and(rp, 3), 5)
            for d in range(D):
                flat = d * LAST_COLS + g * 16
                v = slab_v[flat // 128, pl.ds(flat % 128, 16)]
                plsc.store_scatter(outq_v, [rowv, colbase + d], v)
        pltpu.async_copy(outq_v.at[pl.ds(0, LAST_COLS // 4), :],
                         dst_hbm.at[pl.ds(NJ_FULL * 32, LAST_COLS // 4), :],
                         sem).wait()


def _bpr_body(user_hbm, item_i_hbm, item_j_hbm, euT_hbm, ei_hbm,
              out_i_hbm, out_j_hbm,
              uidx_v, uT_v, urows_v, iidx_v, itrows_v, outv, sem):
    wid = lax.axis_index("s") * NC + lax.axis_index("c")
    lane = lax.iota(jnp.int32, 16)

    def do_chunk(c, _):
        base = wid * ROWS_PER_W + c * CB
        pltpu.sync_copy(user_hbm.at[pl.ds(base, CB)], uidx_v)
        descs = [pltpu.async_copy(euT_hbm.at[d].at[uidx_v], uT_v.at[d], sem)
                 for d in range(D)]
        for dsc in descs:
            dsc.wait()
        for g in range(CB // 16):
            rvec = g * 16 + lane
            for d in range(D):
                plsc.store_scatter(urows_v, [rvec, jnp.full((16,), d, jnp.int32)],
                                   uT_v[d, pl.ds(g * 16, 16)])

        for item_hbm, out_hbm in ((item_i_hbm, out_i_hbm),
                                  (item_j_hbm, out_j_hbm)):
            pltpu.sync_copy(item_hbm.at[:, pl.ds(base, CB)], iidx_v)
            descs = []
            for g in range(NGATHER):
                descs.append(
                    pltpu.async_copy(ei_hbm.at[iidx_v.at[g]],
                                     itrows_v.at[pl.ds(g * IDXW, IDXW)],
                                     sem))
            for dsc in descs:
                dsc.wait()

            def quad_body(q, _):
                accs = [jnp.zeros((16,), jnp.float32) for _ in range(5)]
                for rr in range(4):
                    r = q * 4 + rr
                    u_lo = urows_v[r, pl.ds(0, 16)]
                    u_hi = urows_v[r, pl.ds(16, 16)]
                    for l in range(L):
                        row = l * CB + r
                        p = (itrows_v[row, pl.ds(0, 16)] * u_lo
                             + itrows_v[row, pl.ds(16, 16)] * u_hi)
                        s = jnp.sum(p)
                        vi, lk = divmod(rr * L + l, 16)
                        accs[vi] = jnp.where(lane == lk, s, accs[vi])
                for vi in range(5):
                    outv[pl.ds(q * 80 + vi * 16, 16)] = accs[vi]
                return _

            lax.fori_loop(0, CB // 4, quad_body, 0, unroll=False)
            pltpu.sync_copy(outv, out_hbm.at[pl.ds(base * L, CB * L)])
        return _

    lax.fori_loop(0, NCHUNK, do_chunk, 0, unroll=False)


@jax.jit
def kernel(user, item_i, item_j, embed_user, embed_item):
    user = user.astype(jnp.int32)
    item_i2d = item_i.astype(jnp.int32).T
    item_j2d = item_j.astype(jnp.int32).T
    euT = embed_user.T
    eiT = embed_item.T

    mesh = plsc.VectorSubcoreMesh(core_axis_name="c", subcore_axis_name="s",
                                  num_cores=NC, num_subcores=NS)

    ei_tail = eiT[:, NJ_FULL * 128:].reshape(16, 128)

    relayout = pl.kernel(
        _relayout_body,
        out_type=jax.ShapeDtypeStruct((N_EMB // 4, 128), jnp.float32),
        mesh=mesh,
        compiler_params=pltpu.CompilerParams(needs_layout_passes=False,
                                             use_tc_tiling_on_sc=True),
        scratch_types=[
            pltpu.VMEM((D, 128), jnp.float32),
            pltpu.VMEM((D, 128), jnp.float32),
            pltpu.SemaphoreType.DMA,
        ],
    )
    ei_lin = relayout(eiT, ei_tail).reshape(N_EMB, D)

    f = pl.kernel(
        _bpr_body,
        out_type=(jax.ShapeDtypeStruct((B * L,), jnp.float32),
                  jax.ShapeDtypeStruct((B * L,), jnp.float32)),
        mesh=mesh,
        compiler_params=pltpu.CompilerParams(needs_layout_passes=False,
                                             use_tc_tiling_on_sc=False),
        scratch_types=[
            pltpu.VMEM((CB,), jnp.int32),
            pltpu.VMEM((D, CB), jnp.float32),
            pltpu.VMEM((CB, D), jnp.float32),
            pltpu.VMEM((NGATHER, IDXW), jnp.int32),
            pltpu.VMEM((CB * L, D), jnp.float32),
            pltpu.VMEM((CB * L,), jnp.float32),
            pltpu.SemaphoreType.DMA,
        ],
    )
    out_i, out_j = f(user, item_i2d, item_j2d, euT, ei_lin)
    return out_i.reshape(B, L), out_j.reshape(B, L)
